# baseline (speedup 1.0000x reference)
import functools

import jax
import jax.numpy as jnp
import numpy as np
from jax import lax
from jax.experimental import pallas as pl
from jax.experimental.pallas import tpu as pltpu
from jax.experimental.pallas import tpu_sc as plsc

K = 64
BLK = 8
CAP = 1024
NEG = np.float32(-3.0e38)
LANES = 16
SUP = 25


def _gather1d(x, idx):
    return lax.gather(
        x, idx[:, None],
        lax.GatherDimensionNumbers(
            offset_dims=(), collapsed_slice_dims=(0,), start_index_map=(0,)),
        slice_sizes=(1,),
        mode=lax.GatherScatterMode.PROMISE_IN_BOUNDS)


def _sc_select_kernel(b, v, rows_per_worker, logits_hbm, cv_hbm, ci_hbm,
                      row_v, vbuf, ibuf, off_ref):
    nchv = v // LANES
    nsup = nchv // SUP
    wid = lax.axis_index("s") * 2 + lax.axis_index("c")
    lane = lax.iota(jnp.int32, LANES)

    def row_body(q, _rb):
        row = wid * rows_per_worker + q
        pltpu.sync_copy(logits_hbm.at[row], row_v)

        def init_body(j, _):
            vbuf[pl.ds(j * LANES, LANES)] = jnp.full((LANES,), NEG, jnp.float32)
            ibuf[pl.ds(j * LANES, LANES)] = jnp.zeros((LANES,), jnp.int32)
            return 0
        lax.fori_loop(0, CAP // LANES, init_body, 0)

        def sup_body(s, carry):
            t1, t2, t3, t4 = carry
            acc = row_v[pl.ds(s * (SUP * LANES), LANES)]
            for u in range(1, SUP):
                acc = jnp.maximum(acc, row_v[pl.ds(s * (SUP * LANES) + u * LANES, LANES)])
            n1 = jnp.maximum(t1, acc); acc = jnp.minimum(t1, acc); t1 = n1
            n2 = jnp.maximum(t2, acc); acc = jnp.minimum(t2, acc); t2 = n2
            n3 = jnp.maximum(t3, acc); acc = jnp.minimum(t3, acc); t3 = n3
            t4 = jnp.maximum(t4, acc)
            return t1, t2, t3, t4
        init4 = (jnp.full((LANES,), NEG, jnp.float32),) * 4
        _, _, _, t4 = lax.fori_loop(0, nsup, sup_body, init4)
        t = t4
        for stp in (1, 2, 4, 8):
            t = jnp.minimum(t, _gather1d(t, lane ^ stp))

        off_ref[0] = 0
        GRP = 16

        def grp_body(g, _g):
            base = g * (GRP * LANES)
            vs = [row_v[pl.ds(base + u * LANES, LANES)] for u in range(GRP)]
            mx = vs[0]
            for u in range(1, GRP):
                mx = jnp.maximum(mx, vs[u])
            gm = jnp.where(mx >= t, 1, 0)
            for sh in (1, 2, 4, 8):
                gm = gm | _gather1d(gm, lane ^ sh)

            @pl.when(gm[0] > 0)
            def _():
                for u in range(GRP):
                    val = vs[u]
                    mi = jnp.where(val >= t, 1, 0)
                    sm = mi
                    for sh in (1, 2, 4, 8):
                        sm = sm + _gather1d(sm, lane ^ sh)

                    def inner(_, m):
                        fl = jnp.where(m > 0, lane, LANES - 1)
                        for sh in (1, 2, 4, 8):
                            fl = jnp.minimum(fl, _gather1d(fl, lane ^ sh))
                        vsp = _gather1d(val, fl)
                        isp = (base + u * LANES) + fl
                        off2 = jnp.minimum(off_ref[0], CAP - LANES)
                        vbuf[pl.ds(off2, LANES)] = vsp
                        ibuf[pl.ds(off2, LANES)] = isp
                        off_ref[0] = off2 + 1
                        return jnp.where(lane == fl, 0, m)

                    lax.fori_loop(0, sm[0], inner, mi)
            return 0
        lax.fori_loop(0, nchv // GRP, grp_body, 0)

        pltpu.sync_copy(vbuf, cv_hbm.at[row])
        pltpu.sync_copy(ibuf, ci_hbm.at[row])
        return 0

    lax.fori_loop(0, rows_per_worker, row_body, 0)


def _sc_select(logits):
    b, v = logits.shape
    mesh = plsc.VectorSubcoreMesh(core_axis_name="c", subcore_axis_name="s")
    rows_per_worker = b // 32
    fn = functools.partial(
        pl.kernel,
        mesh=mesh,
        out_type=[
            jax.ShapeDtypeStruct((b, CAP), jnp.float32),
            jax.ShapeDtypeStruct((b, CAP), jnp.int32),
        ],
        scratch_types=[
            pltpu.VMEM((v,), jnp.float32),
            pltpu.VMEM((CAP,), jnp.float32),
            pltpu.VMEM((CAP,), jnp.int32),
            pltpu.SMEM((1,), jnp.int32),
        ],
    )(functools.partial(_sc_select_kernel, b, v, rows_per_worker))
    return fn(logits)


def _threefry_bits_at(pos):
    k1 = jnp.uint32(0)
    k2 = jnp.uint32(1)
    ks0, ks1 = k1, k2
    ks2 = k1 ^ k2 ^ jnp.uint32(0x1BD11BDA)

    def rot(x, r):
        return (x << jnp.uint32(r)) | (x >> jnp.uint32(32 - r))

    def rounds(x0, x1, rots):
        for r in rots:
            x0 = x0 + x1
            x1 = rot(x1, r)
            x1 = x0 ^ x1
        return x0, x1

    ra = (13, 15, 26, 6)
    rb = (17, 29, 16, 24)
    x0 = jnp.zeros_like(pos) + ks0
    x1 = pos + ks1
    x0, x1 = rounds(x0, x1, ra); x0 = x0 + ks1; x1 = x1 + ks2 + jnp.uint32(1)
    x0, x1 = rounds(x0, x1, rb); x0 = x0 + ks2; x1 = x1 + ks0 + jnp.uint32(2)
    x0, x1 = rounds(x0, x1, ra); x0 = x0 + ks0; x1 = x1 + ks1 + jnp.uint32(3)
    x0, x1 = rounds(x0, x1, rb); x0 = x0 + ks1; x1 = x1 + ks2 + jnp.uint32(4)
    x0, x1 = rounds(x0, x1, ra); x0 = x0 + ks2; x1 = x1 + ks0 + jnp.uint32(5)
    return x0 ^ x1


def _gumbel_block(row0, v):
    j = lax.broadcasted_iota(jnp.uint32, (BLK, K), 1)
    b = row0.astype(jnp.uint32) + lax.broadcasted_iota(jnp.uint32, (BLK, K), 0)
    bits = _threefry_bits_at(b * jnp.uint32(v) + j)
    f = lax.bitcast_convert_type(
        (bits >> jnp.uint32(9)) | jnp.uint32(0x3F800000), jnp.float32
    ) - jnp.float32(1.0)
    tiny = jnp.float32(1.1754944e-38)
    u = jnp.maximum(tiny, f * (jnp.float32(1.0) - tiny) + tiny)
    return -jnp.log(-jnp.log(u))


def _tc_sample_kernel(v, logits_ref, cv_ref, ci_ref, temp_ref, topk_ref,
                      topp_ref, minp_ref, tok_ref, lp_ref):
    rtemp = jnp.float32(1.0) / temp_ref[...]
    cv = cv_ref[...] * rtemp
    ci = ci_ref[...]
    m = jnp.max(cv, axis=-1, keepdims=True)
    x = logits_ref[...] * rtemp
    s = jnp.sum(jnp.exp(x - m), axis=-1, keepdims=True)
    p = jnp.exp(cv - m) / s

    def body(k, carry):
        acc_v, acc_i, pw = carry
        mx = jnp.max(pw, axis=-1, keepdims=True)
        ix = jnp.max(jnp.where(pw == mx, ci, -1), axis=-1, keepdims=True)
        pw = jnp.where((pw == mx) & (ci == ix), jnp.float32(-1.0), pw)
        jj = lax.broadcasted_iota(jnp.int32, (BLK, K), 1)
        acc_v = jnp.where(jj == k, mx, acc_v)
        acc_i = jnp.where(jj == k, ix, acc_i)
        return acc_v, acc_i, pw

    topv, topi, _ = lax.fori_loop(
        0, K, body,
        (jnp.zeros((BLK, K), jnp.float32), jnp.zeros((BLK, K), jnp.int32), p),
    )

    cs = topv
    for sh in (1, 2, 4, 8, 16, 32):
        shifted = jnp.concatenate(
            [jnp.zeros((BLK, sh), jnp.float32), cs[:, : K - sh]], axis=1)
        cs = cs + shifted

    jj = lax.broadcasted_iota(jnp.int32, (BLK, K), 1)
    pf = jnp.where(jj >= topk_ref[...], jnp.float32(0.0), topv)
    pf = jnp.where(cs - pf > topp_ref[...], jnp.float32(0.0), pf)
    thr = pf[:, 0:1] * minp_ref[...]
    pf = jnp.where(pf < thr, jnp.float32(0.0), pf)

    row0 = pl.program_id(0) * BLK
    g = _gumbel_block(row0, v)
    score = jnp.where(pf > 0, jnp.log(pf) + g, NEG)
    smax = jnp.max(score, axis=-1, keepdims=True)
    sampled = jnp.min(jnp.where(score == smax, jj, K), axis=-1, keepdims=True)
    onehot = jj == sampled
    tok_ref[...] = jnp.sum(jnp.where(onehot, topi, 0), axis=-1, keepdims=True)
    lp_ref[...] = jnp.sum(
        jnp.where(onehot, jnp.log(topv), jnp.float32(0.0)),
        axis=-1, keepdims=True)


def kernel(logits, temperatures, top_ks, top_ps, min_ps):
    b, v = logits.shape
    cand_v, cand_i = _sc_select(logits)
    grid = b // BLK
    tok, lp = pl.pallas_call(
        functools.partial(_tc_sample_kernel, v),
        grid=(grid,),
        in_specs=[
            pl.BlockSpec((BLK, v), lambda i: (i, 0)),
            pl.BlockSpec((BLK, CAP), lambda i: (i, 0)),
            pl.BlockSpec((BLK, CAP), lambda i: (i, 0)),
            pl.BlockSpec((BLK, 1), lambda i: (i, 0)),
            pl.BlockSpec((BLK, 1), lambda i: (i, 0)),
            pl.BlockSpec((BLK, 1), lambda i: (i, 0)),
            pl.BlockSpec((BLK, 1), lambda i: (i, 0)),
        ],
        out_specs=[
            pl.BlockSpec((BLK, 1), lambda i: (i, 0)),
            pl.BlockSpec((BLK, 1), lambda i: (i, 0)),
        ],
        out_shape=[
            jax.ShapeDtypeStruct((b, 1), jnp.int32),
            jax.ShapeDtypeStruct((b, 1), jnp.float32),
        ],
    )(
        logits,
        cand_v,
        cand_i,
        temperatures,
        top_ks.reshape(b, 1),
        top_ps.reshape(b, 1),
        min_ps.reshape(b, 1),
    )
    return tok[:, 0], lp[:, 0]


# speedup vs baseline: 1.0544x; 1.0544x over previous
import functools

import jax
import jax.numpy as jnp
import numpy as np
from jax import lax
from jax.experimental import pallas as pl
from jax.experimental.pallas import tpu as pltpu
from jax.experimental.pallas import tpu_sc as plsc

K = 64
BLK = 8
CAP = 1024
NEG = np.float32(-3.0e38)
LANES = 16
SUP = 25


def _gather1d(x, idx):
    return lax.gather(
        x, idx[:, None],
        lax.GatherDimensionNumbers(
            offset_dims=(), collapsed_slice_dims=(0,), start_index_map=(0,)),
        slice_sizes=(1,),
        mode=lax.GatherScatterMode.PROMISE_IN_BOUNDS)


def _sc_select_kernel(b, v, rows_per_worker, logits_hbm, cv_hbm, ci_hbm,
                      row_v, vbuf, ibuf, off_ref):
    nchv = v // LANES
    nsup = nchv // SUP
    wid = lax.axis_index("s") * 2 + lax.axis_index("c")
    lane = lax.iota(jnp.int32, LANES)

    def row_body(q, _rb):
        row = wid * rows_per_worker + q
        pltpu.sync_copy(logits_hbm.at[row], row_v)

        def init_body(j, _):
            vbuf[pl.ds(j * LANES, LANES)] = jnp.full((LANES,), NEG, jnp.float32)
            ibuf[pl.ds(j * LANES, LANES)] = jnp.zeros((LANES,), jnp.int32)
            return 0
        lax.fori_loop(0, CAP // LANES, init_body, 0)

        def sup_body(s, carry):
            t1, t2, t3, t4 = carry
            acc = row_v[pl.ds(s * (SUP * LANES), LANES)]
            for u in range(1, SUP):
                acc = jnp.maximum(acc, row_v[pl.ds(s * (SUP * LANES) + u * LANES, LANES)])
            n1 = jnp.maximum(t1, acc); acc = jnp.minimum(t1, acc); t1 = n1
            n2 = jnp.maximum(t2, acc); acc = jnp.minimum(t2, acc); t2 = n2
            n3 = jnp.maximum(t3, acc); acc = jnp.minimum(t3, acc); t3 = n3
            t4 = jnp.maximum(t4, acc)
            return t1, t2, t3, t4
        init4 = (jnp.full((LANES,), NEG, jnp.float32),) * 4
        _, _, _, t4 = lax.fori_loop(0, nsup, sup_body, init4)
        t = t4
        for stp in (1, 2, 4, 8):
            t = jnp.minimum(t, _gather1d(t, lane ^ stp))

        off_ref[0] = 0
        GRP = 10

        def grp_body(g, _g):
            base = g * (GRP * LANES)
            vs = [row_v[pl.ds(base + u * LANES, LANES)] for u in range(GRP)]
            mx = vs[0]
            for u in range(1, GRP):
                mx = jnp.maximum(mx, vs[u])
            gm = jnp.where(mx >= t, 1, 0)
            for sh in (1, 2, 4, 8):
                gm = gm | _gather1d(gm, lane ^ sh)

            @pl.when(gm[0] > 0)
            def _():
                for u in range(GRP):
                    val = vs[u]
                    mi = jnp.where(val >= t, 1, 0)
                    sm = mi
                    for sh in (1, 2, 4, 8):
                        sm = sm + _gather1d(sm, lane ^ sh)

                    def inner(_, m):
                        fl = jnp.where(m > 0, lane, LANES - 1)
                        for sh in (1, 2, 4, 8):
                            fl = jnp.minimum(fl, _gather1d(fl, lane ^ sh))
                        vsp = _gather1d(val, fl)
                        isp = (base + u * LANES) + fl
                        off2 = jnp.minimum(off_ref[0], CAP - LANES)
                        vbuf[pl.ds(off2, LANES)] = vsp
                        ibuf[pl.ds(off2, LANES)] = isp
                        off_ref[0] = off2 + 1
                        return jnp.where(lane == fl, 0, m)

                    lax.fori_loop(0, sm[0], inner, mi)
            return 0
        lax.fori_loop(0, nchv // GRP, grp_body, 0)

        pltpu.sync_copy(vbuf, cv_hbm.at[row])
        pltpu.sync_copy(ibuf, ci_hbm.at[row])
        return 0

    lax.fori_loop(0, rows_per_worker, row_body, 0)


def _sc_select(logits):
    b, v = logits.shape
    mesh = plsc.VectorSubcoreMesh(core_axis_name="c", subcore_axis_name="s")
    rows_per_worker = b // 32
    fn = functools.partial(
        pl.kernel,
        mesh=mesh,
        out_type=[
            jax.ShapeDtypeStruct((b, CAP), jnp.float32),
            jax.ShapeDtypeStruct((b, CAP), jnp.int32),
        ],
        scratch_types=[
            pltpu.VMEM((v,), jnp.float32),
            pltpu.VMEM((CAP,), jnp.float32),
            pltpu.VMEM((CAP,), jnp.int32),
            pltpu.SMEM((1,), jnp.int32),
        ],
    )(functools.partial(_sc_select_kernel, b, v, rows_per_worker))
    return fn(logits)


def _threefry_bits_at(pos):
    k1 = jnp.uint32(0)
    k2 = jnp.uint32(1)
    ks0, ks1 = k1, k2
    ks2 = k1 ^ k2 ^ jnp.uint32(0x1BD11BDA)

    def rot(x, r):
        return (x << jnp.uint32(r)) | (x >> jnp.uint32(32 - r))

    def rounds(x0, x1, rots):
        for r in rots:
            x0 = x0 + x1
            x1 = rot(x1, r)
            x1 = x0 ^ x1
        return x0, x1

    ra = (13, 15, 26, 6)
    rb = (17, 29, 16, 24)
    x0 = jnp.zeros_like(pos) + ks0
    x1 = pos + ks1
    x0, x1 = rounds(x0, x1, ra); x0 = x0 + ks1; x1 = x1 + ks2 + jnp.uint32(1)
    x0, x1 = rounds(x0, x1, rb); x0 = x0 + ks2; x1 = x1 + ks0 + jnp.uint32(2)
    x0, x1 = rounds(x0, x1, ra); x0 = x0 + ks0; x1 = x1 + ks1 + jnp.uint32(3)
    x0, x1 = rounds(x0, x1, rb); x0 = x0 + ks1; x1 = x1 + ks2 + jnp.uint32(4)
    x0, x1 = rounds(x0, x1, ra); x0 = x0 + ks2; x1 = x1 + ks0 + jnp.uint32(5)
    return x0 ^ x1


def _gumbel_block(row0, v):
    j = lax.broadcasted_iota(jnp.uint32, (BLK, K), 1)
    b = row0.astype(jnp.uint32) + lax.broadcasted_iota(jnp.uint32, (BLK, K), 0)
    bits = _threefry_bits_at(b * jnp.uint32(v) + j)
    f = lax.bitcast_convert_type(
        (bits >> jnp.uint32(9)) | jnp.uint32(0x3F800000), jnp.float32
    ) - jnp.float32(1.0)
    tiny = jnp.float32(1.1754944e-38)
    u = jnp.maximum(tiny, f * (jnp.float32(1.0) - tiny) + tiny)
    return -jnp.log(-jnp.log(u))


def _tc_sample_kernel(v, logits_ref, cv_ref, ci_ref, temp_ref, topk_ref,
                      topp_ref, minp_ref, tok_ref, lp_ref):
    rtemp = jnp.float32(1.0) / temp_ref[...]
    cv = cv_ref[...] * rtemp
    ci = ci_ref[...]
    m = jnp.max(cv, axis=-1, keepdims=True)
    x = logits_ref[...] * rtemp
    s = jnp.sum(jnp.exp(x - m), axis=-1, keepdims=True)
    p = jnp.exp(cv - m) / s

    def body(k, carry):
        acc_v, acc_i, pw = carry
        mx = jnp.max(pw, axis=-1, keepdims=True)
        ix = jnp.max(jnp.where(pw == mx, ci, -1), axis=-1, keepdims=True)
        pw = jnp.where((pw == mx) & (ci == ix), jnp.float32(-1.0), pw)
        jj = lax.broadcasted_iota(jnp.int32, (BLK, K), 1)
        acc_v = jnp.where(jj == k, mx, acc_v)
        acc_i = jnp.where(jj == k, ix, acc_i)
        return acc_v, acc_i, pw

    topv, topi, _ = lax.fori_loop(
        0, K, body,
        (jnp.zeros((BLK, K), jnp.float32), jnp.zeros((BLK, K), jnp.int32), p),
    )

    cs = topv
    for sh in (1, 2, 4, 8, 16, 32):
        shifted = jnp.concatenate(
            [jnp.zeros((BLK, sh), jnp.float32), cs[:, : K - sh]], axis=1)
        cs = cs + shifted

    jj = lax.broadcasted_iota(jnp.int32, (BLK, K), 1)
    pf = jnp.where(jj >= topk_ref[...], jnp.float32(0.0), topv)
    pf = jnp.where(cs - pf > topp_ref[...], jnp.float32(0.0), pf)
    thr = pf[:, 0:1] * minp_ref[...]
    pf = jnp.where(pf < thr, jnp.float32(0.0), pf)

    row0 = pl.program_id(0) * BLK
    g = _gumbel_block(row0, v)
    score = jnp.where(pf > 0, jnp.log(pf) + g, NEG)
    smax = jnp.max(score, axis=-1, keepdims=True)
    sampled = jnp.min(jnp.where(score == smax, jj, K), axis=-1, keepdims=True)
    onehot = jj == sampled
    tok_ref[...] = jnp.sum(jnp.where(onehot, topi, 0), axis=-1, keepdims=True)
    lp_ref[...] = jnp.sum(
        jnp.where(onehot, jnp.log(topv), jnp.float32(0.0)),
        axis=-1, keepdims=True)


def kernel(logits, temperatures, top_ks, top_ps, min_ps):
    b, v = logits.shape
    cand_v, cand_i = _sc_select(logits)
    grid = b // BLK
    tok, lp = pl.pallas_call(
        functools.partial(_tc_sample_kernel, v),
        grid=(grid,),
        in_specs=[
            pl.BlockSpec((BLK, v), lambda i: (i, 0)),
            pl.BlockSpec((BLK, CAP), lambda i: (i, 0)),
            pl.BlockSpec((BLK, CAP), lambda i: (i, 0)),
            pl.BlockSpec((BLK, 1), lambda i: (i, 0)),
            pl.BlockSpec((BLK, 1), lambda i: (i, 0)),
            pl.BlockSpec((BLK, 1), lambda i: (i, 0)),
            pl.BlockSpec((BLK, 1), lambda i: (i, 0)),
        ],
        out_specs=[
            pl.BlockSpec((BLK, 1), lambda i: (i, 0)),
            pl.BlockSpec((BLK, 1), lambda i: (i, 0)),
        ],
        out_shape=[
            jax.ShapeDtypeStruct((b, 1), jnp.int32),
            jax.ShapeDtypeStruct((b, 1), jnp.float32),
        ],
    )(
        logits,
        cand_v,
        cand_i,
        temperatures,
        top_ks.reshape(b, 1),
        top_ps.reshape(b, 1),
        min_ps.reshape(b, 1),
    )
    return tok[:, 0], lp[:, 0]


# speedup vs baseline: 1.0782x; 1.0226x over previous
import functools

import jax
import jax.numpy as jnp
import numpy as np
from jax import lax
from jax.experimental import pallas as pl
from jax.experimental.pallas import tpu as pltpu
from jax.experimental.pallas import tpu_sc as plsc

K = 64
BLK = 8
CAP = 1024
NEG = np.float32(-3.0e38)
LANES = 16
SUP = 25


def _gather1d(x, idx):
    return lax.gather(
        x, idx[:, None],
        lax.GatherDimensionNumbers(
            offset_dims=(), collapsed_slice_dims=(0,), start_index_map=(0,)),
        slice_sizes=(1,),
        mode=lax.GatherScatterMode.PROMISE_IN_BOUNDS)


def _sc_select_kernel(b, v, rows_per_worker, logits_hbm, cv_hbm, ci_hbm,
                      row_v, vbuf, ibuf, off_ref):
    nchv = v // LANES
    nsup = nchv // SUP
    wid = lax.axis_index("s") * 2 + lax.axis_index("c")
    lane = lax.iota(jnp.int32, LANES)

    def row_body(q, _rb):
        row = wid * rows_per_worker + q
        pltpu.sync_copy(logits_hbm.at[row], row_v)

        def init_body(j, _):
            vbuf[pl.ds(j * LANES, LANES)] = jnp.full((LANES,), NEG, jnp.float32)
            ibuf[pl.ds(j * LANES, LANES)] = jnp.zeros((LANES,), jnp.int32)
            return 0
        lax.fori_loop(0, CAP // LANES, init_body, 0)

        def sup_body(s, carry):
            t1, t2, t3, t4 = carry
            acc = row_v[pl.ds(s * (SUP * LANES), LANES)]
            for u in range(1, SUP):
                acc = jnp.maximum(acc, row_v[pl.ds(s * (SUP * LANES) + u * LANES, LANES)])
            n1 = jnp.maximum(t1, acc); acc = jnp.minimum(t1, acc); t1 = n1
            n2 = jnp.maximum(t2, acc); acc = jnp.minimum(t2, acc); t2 = n2
            n3 = jnp.maximum(t3, acc); acc = jnp.minimum(t3, acc); t3 = n3
            t4 = jnp.maximum(t4, acc)
            return t1, t2, t3, t4
        init4 = (jnp.full((LANES,), NEG, jnp.float32),) * 4
        _, _, _, t4 = lax.fori_loop(0, nsup, sup_body, init4)
        t = t4
        for stp in (1, 2, 4, 8):
            t = jnp.minimum(t, _gather1d(t, lane ^ stp))

        off_ref[0] = 0
        GRP = 10

        def grp_body(g, _g):
            base = g * (GRP * LANES)
            vs = [row_v[pl.ds(base + u * LANES, LANES)] for u in range(GRP)]
            mx = vs[0]
            for u in range(1, GRP):
                mx = jnp.maximum(mx, vs[u])
            gm = jnp.where(mx >= t, 1, 0)
            for sh in (1, 2, 4, 8):
                gm = gm | _gather1d(gm, lane ^ sh)

            @pl.when(gm[0] > 0)
            def _():
                for u in range(GRP):
                    val = vs[u]
                    mi = jnp.where(val >= t, 1, 0)
                    sm = mi
                    for sh in (1, 2, 4, 8):
                        sm = sm + _gather1d(sm, lane ^ sh)

                    def inner(_, m):
                        fl = jnp.where(m > 0, lane, LANES - 1)
                        for sh in (1, 2, 4, 8):
                            fl = jnp.minimum(fl, _gather1d(fl, lane ^ sh))
                        vsp = _gather1d(val, fl)
                        isp = (base + u * LANES) + fl
                        off2 = jnp.minimum(off_ref[0], CAP - LANES)
                        vbuf[pl.ds(off2, LANES)] = vsp
                        ibuf[pl.ds(off2, LANES)] = isp
                        off_ref[0] = off2 + 1
                        return jnp.where(lane == fl, 0, m)

                    lax.fori_loop(0, sm[0], inner, mi)
            return 0
        lax.fori_loop(0, nchv // GRP, grp_body, 0)

        pltpu.sync_copy(vbuf, cv_hbm.at[row])
        pltpu.sync_copy(ibuf, ci_hbm.at[row])
        return 0

    lax.fori_loop(0, rows_per_worker, row_body, 0)


def _sc_select(logits):
    b, v = logits.shape
    mesh = plsc.VectorSubcoreMesh(core_axis_name="c", subcore_axis_name="s")
    rows_per_worker = b // 32
    fn = functools.partial(
        pl.kernel,
        mesh=mesh,
        out_type=[
            jax.ShapeDtypeStruct((b, CAP), jnp.float32),
            jax.ShapeDtypeStruct((b, CAP), jnp.int32),
        ],
        scratch_types=[
            pltpu.VMEM((v,), jnp.float32),
            pltpu.VMEM((CAP,), jnp.float32),
            pltpu.VMEM((CAP,), jnp.int32),
            pltpu.SMEM((1,), jnp.int32),
        ],
    )(functools.partial(_sc_select_kernel, b, v, rows_per_worker))
    return fn(logits)


def _threefry_bits_at(pos):
    k1 = jnp.uint32(0)
    k2 = jnp.uint32(1)
    ks0, ks1 = k1, k2
    ks2 = k1 ^ k2 ^ jnp.uint32(0x1BD11BDA)

    def rot(x, r):
        return (x << jnp.uint32(r)) | (x >> jnp.uint32(32 - r))

    def rounds(x0, x1, rots):
        for r in rots:
            x0 = x0 + x1
            x1 = rot(x1, r)
            x1 = x0 ^ x1
        return x0, x1

    ra = (13, 15, 26, 6)
    rb = (17, 29, 16, 24)
    x0 = jnp.zeros_like(pos) + ks0
    x1 = pos + ks1
    x0, x1 = rounds(x0, x1, ra); x0 = x0 + ks1; x1 = x1 + ks2 + jnp.uint32(1)
    x0, x1 = rounds(x0, x1, rb); x0 = x0 + ks2; x1 = x1 + ks0 + jnp.uint32(2)
    x0, x1 = rounds(x0, x1, ra); x0 = x0 + ks0; x1 = x1 + ks1 + jnp.uint32(3)
    x0, x1 = rounds(x0, x1, rb); x0 = x0 + ks1; x1 = x1 + ks2 + jnp.uint32(4)
    x0, x1 = rounds(x0, x1, ra); x0 = x0 + ks2; x1 = x1 + ks0 + jnp.uint32(5)
    return x0 ^ x1


def _gumbel_block(row0, v):
    j = lax.broadcasted_iota(jnp.uint32, (BLK, K), 1)
    b = row0.astype(jnp.uint32) + lax.broadcasted_iota(jnp.uint32, (BLK, K), 0)
    bits = _threefry_bits_at(b * jnp.uint32(v) + j)
    f = lax.bitcast_convert_type(
        (bits >> jnp.uint32(9)) | jnp.uint32(0x3F800000), jnp.float32
    ) - jnp.float32(1.0)
    tiny = jnp.float32(1.1754944e-38)
    u = jnp.maximum(tiny, f * (jnp.float32(1.0) - tiny) + tiny)
    return -jnp.log(-jnp.log(u))


def _tc_stats_kernel(logits_ref, temp_ref, m_ref, s_ref):
    x = logits_ref[...] * (jnp.float32(1.0) / temp_ref[...])
    m = jnp.max(x, axis=-1, keepdims=True)
    m_ref[...] = m
    s_ref[...] = jnp.sum(jnp.exp(x - m), axis=-1, keepdims=True)


def _tc_sample_kernel(v, cv_ref, ci_ref, m_ref, s_ref, temp_ref, topk_ref,
                      topp_ref, minp_ref, tok_ref, lp_ref):
    rtemp = jnp.float32(1.0) / temp_ref[...]
    cv = cv_ref[...] * rtemp
    ci = ci_ref[...]
    m = m_ref[...]
    s = s_ref[...]
    p = jnp.exp(cv - m) / s

    def body(k, carry):
        acc_v, acc_i, pw = carry
        mx = jnp.max(pw, axis=-1, keepdims=True)
        ix = jnp.max(jnp.where(pw == mx, ci, -1), axis=-1, keepdims=True)
        pw = jnp.where((pw == mx) & (ci == ix), jnp.float32(-1.0), pw)
        jj = lax.broadcasted_iota(jnp.int32, (BLK, K), 1)
        acc_v = jnp.where(jj == k, mx, acc_v)
        acc_i = jnp.where(jj == k, ix, acc_i)
        return acc_v, acc_i, pw

    topv, topi, _ = lax.fori_loop(
        0, K, body,
        (jnp.zeros((BLK, K), jnp.float32), jnp.zeros((BLK, K), jnp.int32), p),
    )

    cs = topv
    for sh in (1, 2, 4, 8, 16, 32):
        shifted = jnp.concatenate(
            [jnp.zeros((BLK, sh), jnp.float32), cs[:, : K - sh]], axis=1)
        cs = cs + shifted

    jj = lax.broadcasted_iota(jnp.int32, (BLK, K), 1)
    pf = jnp.where(jj >= topk_ref[...], jnp.float32(0.0), topv)
    pf = jnp.where(cs - pf > topp_ref[...], jnp.float32(0.0), pf)
    thr = pf[:, 0:1] * minp_ref[...]
    pf = jnp.where(pf < thr, jnp.float32(0.0), pf)

    row0 = pl.program_id(0) * BLK
    g = _gumbel_block(row0, v)
    score = jnp.where(pf > 0, jnp.log(pf) + g, NEG)
    smax = jnp.max(score, axis=-1, keepdims=True)
    sampled = jnp.min(jnp.where(score == smax, jj, K), axis=-1, keepdims=True)
    onehot = jj == sampled
    tok_ref[...] = jnp.sum(jnp.where(onehot, topi, 0), axis=-1, keepdims=True)
    lp_ref[...] = jnp.sum(
        jnp.where(onehot, jnp.log(topv), jnp.float32(0.0)),
        axis=-1, keepdims=True)


def kernel(logits, temperatures, top_ks, top_ps, min_ps):
    b, v = logits.shape
    grid = b // BLK
    cand_v, cand_i = _sc_select(logits)
    m, s = pl.pallas_call(
        _tc_stats_kernel,
        grid=(grid,),
        in_specs=[
            pl.BlockSpec((BLK, v), lambda i: (i, 0)),
            pl.BlockSpec((BLK, 1), lambda i: (i, 0)),
        ],
        out_specs=[
            pl.BlockSpec((BLK, 1), lambda i: (i, 0)),
            pl.BlockSpec((BLK, 1), lambda i: (i, 0)),
        ],
        out_shape=[
            jax.ShapeDtypeStruct((b, 1), jnp.float32),
            jax.ShapeDtypeStruct((b, 1), jnp.float32),
        ],
    )(logits, temperatures)
    tok, lp = pl.pallas_call(
        functools.partial(_tc_sample_kernel, v),
        grid=(grid,),
        in_specs=[
            pl.BlockSpec((BLK, CAP), lambda i: (i, 0)),
            pl.BlockSpec((BLK, CAP), lambda i: (i, 0)),
            pl.BlockSpec((BLK, 1), lambda i: (i, 0)),
            pl.BlockSpec((BLK, 1), lambda i: (i, 0)),
            pl.BlockSpec((BLK, 1), lambda i: (i, 0)),
            pl.BlockSpec((BLK, 1), lambda i: (i, 0)),
            pl.BlockSpec((BLK, 1), lambda i: (i, 0)),
            pl.BlockSpec((BLK, 1), lambda i: (i, 0)),
        ],
        out_specs=[
            pl.BlockSpec((BLK, 1), lambda i: (i, 0)),
            pl.BlockSpec((BLK, 1), lambda i: (i, 0)),
        ],
        out_shape=[
            jax.ShapeDtypeStruct((b, 1), jnp.int32),
            jax.ShapeDtypeStruct((b, 1), jnp.float32),
        ],
    )(
        cand_v,
        cand_i,
        m,
        s,
        temperatures,
        top_ks.reshape(b, 1),
        top_ps.reshape(b, 1),
        min_ps.reshape(b, 1),
    )
    return tok[:, 0], lp[:, 0]
